# Initial kernel scaffold; baseline (speedup 1.0000x reference)
import jax
import jax.numpy as jnp
from jax.experimental import pallas as pl


def kernel(neighbor_index, vertices, feature_map, weights, bias, directions, distance_w):
    raise NotImplementedError("write your pallas kernel here")


# speedup vs baseline: 16.7936x; 16.7936x over previous
import functools

import jax
import jax.numpy as jnp
from jax import lax
from jax.experimental import pallas as pl
from jax.experimental.pallas import tpu as pltpu
from jax.experimental.pallas import tpu_sc as plsc

BS, V, N = 2, 10000, 16
IN_C, OUT_C, SUP = 128, 128, 1
TOT = BS * V
L = 16
NC, NS = 2, 16
NW = NC * NS
TOTP = 20480
W_PER = TOTP // NW
C = 8
NCHUNK = W_PER // C
CCH = OUT_C // L
VPAD = 16


def _mm_body(x_ref, w_ref, b_ref, fc_ref, fs_ref):
    y = jnp.dot(x_ref[...], w_ref[...], preferred_element_type=jnp.float32)
    y = y + b_ref[...]
    fc_ref[...] = y[:, :OUT_C]
    fs_ref[...] = y[:, OUT_C:]


def _matmul(fm, w, b2d):
    BLK = 2048
    return pl.pallas_call(
        _mm_body,
        grid=(TOTP // BLK,),
        in_specs=[
            pl.BlockSpec((BLK, IN_C), lambda i: (i, 0)),
            pl.BlockSpec((IN_C, (SUP + 1) * OUT_C), lambda i: (0, 0)),
            pl.BlockSpec((1, (SUP + 1) * OUT_C), lambda i: (0, 0)),
        ],
        out_specs=[
            pl.BlockSpec((BLK, OUT_C), lambda i: (i, 0)),
            pl.BlockSpec((BLK, OUT_C), lambda i: (i, 0)),
        ],
        out_shape=[
            jax.ShapeDtypeStruct((TOTP, OUT_C), jnp.float32),
            jax.ShapeDtypeStruct((TOTP, OUT_C), jnp.float32),
        ],
    )(fm, w, b2d)


def _rsqrt(x):
    i = lax.bitcast_convert_type(x, jnp.int32)
    i = jnp.int32(0x5F3759DF) - lax.shift_right_logical(i, 1)
    y = lax.bitcast_convert_type(i, jnp.float32)
    for _ in range(3):
        y = y * (1.5 - 0.5 * x * y * y)
    return y


def _sc_body(idx_hbm, vp_hbm, vx_hbm, vy_hbm, vz_hbm, fs_hbm, fc_hbm,
             dirs_hbm, dw_hbm, out_hbm,
             idx_v, npx_v, npy_v, npz_v, fsr_v, sv_v, fc_v, out_v, s_v, dw_v,
             sem_p, sem_f):
    wid = lax.axis_index("s") * NC + lax.axis_index("c")

    pltpu.sync_copy(dirs_hbm, s_v)
    pltpu.sync_copy(dw_hbm, dw_v)
    for cc in range(CCH):
        sl = pl.ds(cc * L, L)
        a = s_v[0, sl]
        b = s_v[1, sl]
        c = s_v[2, sl]
        inv = _rsqrt(jnp.maximum(a * a + b * b + c * c, 1e-24))
        s_v[0, sl] = a * inv
        s_v[1, sl] = b * inv
        s_v[2, sl] = c * inv

    def chunk_body(ci, carry):
        row0 = wid * W_PER + ci * C
        pltpu.sync_copy(idx_hbm.at[pl.ds(row0 * N, C * N)], idx_v)
        gx = pltpu.async_copy(vx_hbm.at[idx_v], npx_v, sem_p)
        gy = pltpu.async_copy(vy_hbm.at[idx_v], npy_v, sem_p)
        gz = pltpu.async_copy(vz_hbm.at[idx_v], npz_v, sem_p)
        gf = pltpu.async_copy(fs_hbm.at[idx_v], fsr_v, sem_f)
        pltpu.sync_copy(vp_hbm.at[pl.ds(row0, C)], sv_v)
        pltpu.sync_copy(fc_hbm.at[pl.ds(row0, C)], fc_v)
        gx.wait()
        gy.wait()
        gz.wait()
        gf.wait()

        def vert_body(v, vcarry):
            srow = sv_v[v, :]
            nsl = pl.ds(v * N, N)
            dx = npx_v[nsl] - srow[0]
            dy = npy_v[nsl] - srow[1]
            dz = npz_v[nsl] - srow[2]
            d2 = dx * dx + dy * dy + dz * dz
            inv = _rsqrt(jnp.maximum(d2, 1e-24))
            dist = d2 * inv
            maxd = dist[0]
            for n in range(1, N):
                maxd = jnp.maximum(maxd, dist[n])
            dnx = dx * inv
            dny = dy * inv
            dnz = dz * inv
            accs = [jnp.full((L,), -jnp.inf, jnp.float32)] * CCH
            svec = [s_v[r, pl.ds(cc * L, L)]
                    for r in range(3) for cc in range(CCH)]
            for n in range(N):
                xn = dnx[n]
                yn = dny[n]
                zn = dnz[n]
                for cc in range(CCH):
                    sl = pl.ds(cc * L, L)
                    theta = jnp.maximum(
                        xn * svec[cc] + yn * svec[CCH + cc]
                        + zn * svec[2 * CCH + cc], 0.0)
                    accs[cc] = jnp.maximum(accs[cc],
                                           theta * fsr_v[v * N + n, sl])
            for cc in range(CCH):
                sl = pl.ds(cc * L, L)
                dterm = jnp.maximum(maxd * dw_v[0, sl], 0.0)
                out_v[v, sl] = fc_v[v, sl] + accs[cc] + dterm
            return vcarry

        lax.fori_loop(0, C, vert_body, 0)
        pltpu.sync_copy(out_v, out_hbm.at[pl.ds(row0, C)])
        return carry

    lax.fori_loop(0, NCHUNK, chunk_body, 0)


_sc_kernel = functools.partial(
    pl.kernel,
    mesh=plsc.VectorSubcoreMesh(core_axis_name="c", subcore_axis_name="s"),
    out_type=jax.ShapeDtypeStruct((TOTP, OUT_C), jnp.float32),
    scratch_types=[
        pltpu.VMEM((C * N,), jnp.int32),
        pltpu.VMEM((C * N,), jnp.float32),
        pltpu.VMEM((C * N,), jnp.float32),
        pltpu.VMEM((C * N,), jnp.float32),
        pltpu.VMEM((C * N, OUT_C), jnp.float32),
        pltpu.VMEM((C, VPAD), jnp.float32),
        pltpu.VMEM((C, OUT_C), jnp.float32),
        pltpu.VMEM((C, OUT_C), jnp.float32),
        pltpu.VMEM((3, OUT_C), jnp.float32),
        pltpu.VMEM((1, OUT_C), jnp.float32),
        pltpu.SemaphoreType.DMA,
        pltpu.SemaphoreType.DMA,
    ],
)(_sc_body)


def kernel(neighbor_index, vertices, feature_map, weights, bias, directions,
           distance_w):
    nidx = neighbor_index.astype(jnp.int32)
    offs = (jnp.arange(BS, dtype=jnp.int32) * V)[:, None, None]
    idx_flat = jnp.pad((nidx + offs).reshape(TOT * N),
                       (0, (TOTP - TOT) * N))
    vflat = vertices.reshape(TOT, 3)
    vp = jnp.pad(vflat, ((0, TOTP - TOT), (0, VPAD - 3)))
    vx = vp[:, 0]
    vy = vp[:, 1]
    vz = vp[:, 2]
    fm = jnp.pad(feature_map.reshape(TOT, IN_C), ((0, TOTP - TOT), (0, 0)))
    fc, fs = _matmul(fm, weights, bias.reshape(1, (SUP + 1) * OUT_C))
    out = _sc_kernel(idx_flat, vp, vx, vy, vz, fs, fc, directions, distance_w)
    return out[:TOT].reshape(BS, V, OUT_C)


# speedup vs baseline: 19.6798x; 1.1719x over previous
import functools

import jax
import jax.numpy as jnp
from jax import lax
from jax.experimental import pallas as pl
from jax.experimental.pallas import tpu as pltpu
from jax.experimental.pallas import tpu_sc as plsc

BS, V, N = 2, 10000, 16
IN_C, OUT_C, SUP = 128, 128, 1
TOT = BS * V
L = 16
NC, NS = 2, 16
NW = NC * NS
TOTP = 20480
W_PER = TOTP // NW
C = 8
NCHUNK = W_PER // C
CCH = OUT_C // L
VPAD = 16


def _mm_body(x_ref, w_ref, b_ref, fc_ref, fs_ref):
    y = jnp.dot(x_ref[...], w_ref[...], preferred_element_type=jnp.float32)
    y = y + b_ref[...]
    fc_ref[...] = y[:, :OUT_C]
    fs_ref[...] = y[:, OUT_C:]


def _matmul(fm, w, b2d):
    BLK = 2048
    return pl.pallas_call(
        _mm_body,
        grid=(TOTP // BLK,),
        in_specs=[
            pl.BlockSpec((BLK, IN_C), lambda i: (i, 0)),
            pl.BlockSpec((IN_C, (SUP + 1) * OUT_C), lambda i: (0, 0)),
            pl.BlockSpec((1, (SUP + 1) * OUT_C), lambda i: (0, 0)),
        ],
        out_specs=[
            pl.BlockSpec((BLK, OUT_C), lambda i: (i, 0)),
            pl.BlockSpec((BLK, OUT_C), lambda i: (i, 0)),
        ],
        out_shape=[
            jax.ShapeDtypeStruct((TOTP, OUT_C), jnp.float32),
            jax.ShapeDtypeStruct((TOTP, OUT_C), jnp.float32),
        ],
    )(fm, w, b2d)


def _rsqrt(x):
    i = lax.bitcast_convert_type(x, jnp.int32)
    i = jnp.int32(0x5F3759DF) - lax.shift_right_logical(i, 1)
    y = lax.bitcast_convert_type(i, jnp.float32)
    for _ in range(3):
        y = y * (1.5 - 0.5 * x * y * y)
    return y


def _sc_body(idx_hbm, vp_hbm, vx_hbm, vy_hbm, vz_hbm, fs_hbm, fc_hbm,
             dirs_hbm, dw_hbm, out_hbm,
             idx_a, npx_a, npy_a, npz_a, fsr_a, sv_a, fc_a, out_a,
             idx_b, npx_b, npy_b, npz_b, fsr_b, sv_b, fc_b, out_b,
             s_v, dw_v, semp_a, semf_a, semp_b, semf_b):
    wid = lax.axis_index("s") * NC + lax.axis_index("c")
    bufs = ((idx_a, npx_a, npy_a, npz_a, fsr_a, sv_a, fc_a, out_a,
             semp_a, semf_a),
            (idx_b, npx_b, npy_b, npz_b, fsr_b, sv_b, fc_b, out_b,
             semp_b, semf_b))

    pltpu.sync_copy(dirs_hbm, s_v)
    pltpu.sync_copy(dw_hbm, dw_v)
    for cc in range(CCH):
        sl = pl.ds(cc * L, L)
        a = s_v[0, sl]
        b = s_v[1, sl]
        c = s_v[2, sl]
        inv = _rsqrt(jnp.maximum(a * a + b * b + c * c, 1e-24))
        s_v[0, sl] = a * inv
        s_v[1, sl] = b * inv
        s_v[2, sl] = c * inv
    svec = [s_v[r, pl.ds(cc * L, L)] for r in range(3) for cc in range(CCH)]
    dwv = [dw_v[0, pl.ds(cc * L, L)] for cc in range(CCH)]

    def issue(ci, buf):
        idx_v, npx_v, npy_v, npz_v, fsr_v, sv_v, fc_v, _, semp, semf = buf
        row0 = wid * W_PER + ci * C
        pltpu.sync_copy(idx_hbm.at[pl.ds(row0 * N, C * N)], idx_v)
        pltpu.async_copy(vx_hbm.at[idx_v], npx_v, semp)
        pltpu.async_copy(vy_hbm.at[idx_v], npy_v, semp)
        pltpu.async_copy(vz_hbm.at[idx_v], npz_v, semp)
        pltpu.async_copy(fs_hbm.at[idx_v], fsr_v, semf)
        pltpu.sync_copy(vp_hbm.at[pl.ds(row0, C)], sv_v)
        pltpu.sync_copy(fc_hbm.at[pl.ds(row0, C)], fc_v)

    def drain(buf):
        idx_v, npx_v, npy_v, npz_v, fsr_v, _, _, _, semp, semf = buf
        pltpu.make_async_copy(vx_hbm.at[idx_v], npx_v, semp).wait()
        pltpu.make_async_copy(vy_hbm.at[idx_v], npy_v, semp).wait()
        pltpu.make_async_copy(vz_hbm.at[idx_v], npz_v, semp).wait()
        pltpu.make_async_copy(fs_hbm.at[idx_v], fsr_v, semf).wait()

    def compute(ci, buf):
        _, npx_v, npy_v, npz_v, fsr_v, sv_v, fc_v, out_v, _, _ = buf
        row0 = wid * W_PER + ci * C

        def vert_body(v, vcarry):
            srow = sv_v[v, :]
            nsl = pl.ds(v * N, N)
            dx = npx_v[nsl] - srow[0]
            dy = npy_v[nsl] - srow[1]
            dz = npz_v[nsl] - srow[2]
            d2 = dx * dx + dy * dy + dz * dz
            inv = _rsqrt(jnp.maximum(d2, 1e-24))
            dist = d2 * inv
            maxd = dist[0]
            for n in range(1, N):
                maxd = jnp.maximum(maxd, dist[n])
            dnx = dx * inv
            dny = dy * inv
            dnz = dz * inv
            accs = [jnp.full((L,), -jnp.inf, jnp.float32)] * CCH
            for n in range(N):
                xn = dnx[n]
                yn = dny[n]
                zn = dnz[n]
                for cc in range(CCH):
                    sl = pl.ds(cc * L, L)
                    theta = jnp.maximum(
                        xn * svec[cc] + yn * svec[CCH + cc]
                        + zn * svec[2 * CCH + cc], 0.0)
                    accs[cc] = jnp.maximum(accs[cc],
                                           theta * fsr_v[v * N + n, sl])
            for cc in range(CCH):
                sl = pl.ds(cc * L, L)
                dterm = jnp.maximum(maxd * dwv[cc], 0.0)
                out_v[v, sl] = fc_v[v, sl] + accs[cc] + dterm
            return vcarry

        lax.fori_loop(0, C, vert_body, 0)
        pltpu.sync_copy(out_v, out_hbm.at[pl.ds(row0, C)])

    issue(0, bufs[0])

    def pair_body(g, carry):
        c0 = 2 * g
        drain(bufs[0])
        issue(c0 + 1, bufs[1])
        compute(c0, bufs[0])
        drain(bufs[1])
        nxt = jnp.where(c0 + 2 < NCHUNK, c0 + 2, 0)
        issue(nxt, bufs[0])
        compute(c0 + 1, bufs[1])
        return carry

    lax.fori_loop(0, NCHUNK // 2, pair_body, 0)
    drain(bufs[0])


_sc_kernel = functools.partial(
    pl.kernel,
    mesh=plsc.VectorSubcoreMesh(core_axis_name="c", subcore_axis_name="s"),
    out_type=jax.ShapeDtypeStruct((TOTP, OUT_C), jnp.float32),
    scratch_types=(
        [pltpu.VMEM((C * N,), jnp.int32),
         pltpu.VMEM((C * N,), jnp.float32),
         pltpu.VMEM((C * N,), jnp.float32),
         pltpu.VMEM((C * N,), jnp.float32),
         pltpu.VMEM((C * N, OUT_C), jnp.float32),
         pltpu.VMEM((C, VPAD), jnp.float32),
         pltpu.VMEM((C, OUT_C), jnp.float32),
         pltpu.VMEM((C, OUT_C), jnp.float32)] * 2
        + [pltpu.VMEM((3, OUT_C), jnp.float32),
           pltpu.VMEM((1, OUT_C), jnp.float32),
           pltpu.SemaphoreType.DMA,
           pltpu.SemaphoreType.DMA,
           pltpu.SemaphoreType.DMA,
           pltpu.SemaphoreType.DMA]
    ),
)(_sc_body)


def kernel(neighbor_index, vertices, feature_map, weights, bias, directions,
           distance_w):
    nidx = neighbor_index.astype(jnp.int32)
    offs = (jnp.arange(BS, dtype=jnp.int32) * V)[:, None, None]
    idx_flat = jnp.pad((nidx + offs).reshape(TOT * N),
                       (0, (TOTP - TOT) * N))
    vflat = vertices.reshape(TOT, 3)
    vp = jnp.pad(vflat, ((0, TOTP - TOT), (0, VPAD - 3)))
    vx = vp[:, 0]
    vy = vp[:, 1]
    vz = vp[:, 2]
    fm = jnp.pad(feature_map.reshape(TOT, IN_C), ((0, TOTP - TOT), (0, 0)))
    fc, fs = _matmul(fm, weights, bias.reshape(1, (SUP + 1) * OUT_C))
    out = _sc_kernel(idx_flat, vp, vx, vy, vz, fs, fc, directions, distance_w)
    return out[:TOT].reshape(BS, V, OUT_C)


# speedup vs baseline: 22.0316x; 1.1195x over previous
import functools

import jax
import jax.numpy as jnp
from jax import lax
from jax.experimental import pallas as pl
from jax.experimental.pallas import tpu as pltpu
from jax.experimental.pallas import tpu_sc as plsc

BS, V, N = 2, 10000, 16
IN_C, OUT_C, SUP = 128, 128, 1
TOT = BS * V
L = 16
NC, NS = 2, 16
NW = NC * NS
TOTP = 20480
W_PER = TOTP // NW
C = 8
NCHUNK = W_PER // C
CCH = OUT_C // L
VPAD = 16


def _mm_body(x_ref, w_ref, b_ref, fc_ref, fs_ref):
    y = jnp.dot(x_ref[...], w_ref[...], preferred_element_type=jnp.float32)
    y = y + b_ref[...]
    fc_ref[...] = y[:, :OUT_C]
    fs_ref[...] = y[:, OUT_C:]


def _matmul(fm, w, b2d):
    BLK = 2048
    return pl.pallas_call(
        _mm_body,
        grid=(TOTP // BLK,),
        in_specs=[
            pl.BlockSpec((BLK, IN_C), lambda i: (i, 0)),
            pl.BlockSpec((IN_C, (SUP + 1) * OUT_C), lambda i: (0, 0)),
            pl.BlockSpec((1, (SUP + 1) * OUT_C), lambda i: (0, 0)),
        ],
        out_specs=[
            pl.BlockSpec((BLK, OUT_C), lambda i: (i, 0)),
            pl.BlockSpec((BLK, OUT_C), lambda i: (i, 0)),
        ],
        out_shape=[
            jax.ShapeDtypeStruct((TOTP, OUT_C), jnp.float32),
            jax.ShapeDtypeStruct((TOTP, OUT_C), jnp.float32),
        ],
    )(fm, w, b2d)


def _rsqrt(x):
    i = lax.bitcast_convert_type(x, jnp.int32)
    i = jnp.int32(0x5F3759DF) - lax.shift_right_logical(i, 1)
    y = lax.bitcast_convert_type(i, jnp.float32)
    for _ in range(3):
        y = y * (1.5 - 0.5 * x * y * y)
    return y


def _sc_body(idx_hbm, vp_hbm, vx_hbm, vy_hbm, vz_hbm, fs_hbm, fc_hbm,
             dirs_hbm, dw_hbm, out_hbm,
             idx_a, npx_a, npy_a, npz_a, fsr_a, sv_a, fc_a, out_a,
             idx_b, npx_b, npy_b, npz_b, fsr_b, sv_b, fc_b, out_b,
             s_v, dw_v, semp_a, semf_a, semi_a, semo_a,
             semp_b, semf_b, semi_b, semo_b):
    wid = lax.axis_index("s") * NC + lax.axis_index("c")
    bufs = ((idx_a, npx_a, npy_a, npz_a, fsr_a, sv_a, fc_a, out_a,
             semp_a, semf_a, semi_a, semo_a),
            (idx_b, npx_b, npy_b, npz_b, fsr_b, sv_b, fc_b, out_b,
             semp_b, semf_b, semi_b, semo_b))

    pltpu.sync_copy(dirs_hbm, s_v)
    pltpu.sync_copy(dw_hbm, dw_v)
    for cc in range(CCH):
        sl = pl.ds(cc * L, L)
        a = s_v[0, sl]
        b = s_v[1, sl]
        c = s_v[2, sl]
        inv = _rsqrt(jnp.maximum(a * a + b * b + c * c, 1e-24))
        s_v[0, sl] = a * inv
        s_v[1, sl] = b * inv
        s_v[2, sl] = c * inv
    svec = [s_v[r, pl.ds(cc * L, L)] for r in range(3) for cc in range(CCH)]
    dwv = [dw_v[0, pl.ds(cc * L, L)] for cc in range(CCH)]

    def start_idx(ci, buf):
        idx_v, semi = buf[0], buf[10]
        row0 = wid * W_PER + ci * C
        pltpu.async_copy(idx_hbm.at[pl.ds(row0 * N, C * N)], idx_v, semi)

    def wait_idx(buf):
        idx_v, semi = buf[0], buf[10]
        pltpu.make_async_copy(idx_hbm.at[pl.ds(0, C * N)], idx_v, semi).wait()

    def issue(ci, buf):
        idx_v, npx_v, npy_v, npz_v, fsr_v, sv_v, fc_v = buf[:7]
        semp, semf = buf[8], buf[9]
        row0 = wid * W_PER + ci * C
        pltpu.async_copy(vx_hbm.at[idx_v], npx_v, semp)
        pltpu.async_copy(vy_hbm.at[idx_v], npy_v, semp)
        pltpu.async_copy(vz_hbm.at[idx_v], npz_v, semp)
        pltpu.async_copy(fs_hbm.at[idx_v], fsr_v, semf)
        pltpu.async_copy(vp_hbm.at[pl.ds(row0, C)], sv_v, semp)
        pltpu.async_copy(fc_hbm.at[pl.ds(row0, C)], fc_v, semf)

    def drain(buf):
        idx_v, npx_v, npy_v, npz_v, fsr_v, sv_v, fc_v = buf[:7]
        semp, semf = buf[8], buf[9]
        pltpu.make_async_copy(vx_hbm.at[idx_v], npx_v, semp).wait()
        pltpu.make_async_copy(vy_hbm.at[idx_v], npy_v, semp).wait()
        pltpu.make_async_copy(vz_hbm.at[idx_v], npz_v, semp).wait()
        pltpu.make_async_copy(vp_hbm.at[pl.ds(0, C)], sv_v, semp).wait()
        pltpu.make_async_copy(fs_hbm.at[idx_v], fsr_v, semf).wait()
        pltpu.make_async_copy(fc_hbm.at[pl.ds(0, C)], fc_v, semf).wait()

    def drain_out(buf):
        out_v, semo = buf[7], buf[11]
        pltpu.make_async_copy(out_v, out_hbm.at[pl.ds(0, C)], semo).wait()

    def compute(ci, buf):
        _, npx_v, npy_v, npz_v, fsr_v, sv_v, fc_v, out_v = buf[:8]
        semo = buf[11]
        row0 = wid * W_PER + ci * C

        def vert_body(v, vcarry):
            srow = sv_v[v, :]
            nsl = pl.ds(v * N, N)
            dx = npx_v[nsl] - srow[0]
            dy = npy_v[nsl] - srow[1]
            dz = npz_v[nsl] - srow[2]
            d2 = dx * dx + dy * dy + dz * dz
            inv = _rsqrt(jnp.maximum(d2, 1e-24))
            dist = d2 * inv
            maxd = dist[0]
            for n in range(1, N):
                maxd = jnp.maximum(maxd, dist[n])
            dnx = dx * inv
            dny = dy * inv
            dnz = dz * inv
            accs = [jnp.full((L,), -jnp.inf, jnp.float32)] * CCH
            for n in range(N):
                xn = dnx[n]
                yn = dny[n]
                zn = dnz[n]
                for cc in range(CCH):
                    sl = pl.ds(cc * L, L)
                    theta = jnp.maximum(
                        xn * svec[cc] + yn * svec[CCH + cc]
                        + zn * svec[2 * CCH + cc], 0.0)
                    accs[cc] = jnp.maximum(accs[cc],
                                           theta * fsr_v[v * N + n, sl])
            for cc in range(CCH):
                sl = pl.ds(cc * L, L)
                dterm = jnp.maximum(maxd * dwv[cc], 0.0)
                out_v[v, sl] = fc_v[v, sl] + accs[cc] + dterm
            return vcarry

        lax.fori_loop(0, C, vert_body, 0)
        pltpu.async_copy(out_v, out_hbm.at[pl.ds(row0, C)], semo)

    pltpu.sync_copy(idx_hbm.at[pl.ds(wid * W_PER * N, C * N)], bufs[0][0])
    issue(0, bufs[0])
    start_idx(1, bufs[1])
    pltpu.async_copy(bufs[0][7], out_hbm.at[pl.ds(wid * W_PER, C)],
                     bufs[0][11])
    pltpu.async_copy(bufs[1][7], out_hbm.at[pl.ds(wid * W_PER + C, C)],
                     bufs[1][11])

    def pair_body(g, carry):
        c0 = 2 * g
        n2 = jnp.where(c0 + 2 < NCHUNK, c0 + 2, 0)
        n3 = jnp.where(c0 + 3 < NCHUNK, c0 + 3, 0)
        drain(bufs[0])
        wait_idx(bufs[1])
        issue(c0 + 1, bufs[1])
        start_idx(n2, bufs[0])
        drain_out(bufs[0])
        compute(c0, bufs[0])
        drain(bufs[1])
        wait_idx(bufs[0])
        issue(n2, bufs[0])
        start_idx(n3, bufs[1])
        drain_out(bufs[1])
        compute(c0 + 1, bufs[1])
        return carry

    lax.fori_loop(0, NCHUNK // 2, pair_body, 0)
    wait_idx(bufs[1])
    drain(bufs[0])
    drain_out(bufs[0])
    drain_out(bufs[1])


_sc_kernel = functools.partial(
    pl.kernel,
    mesh=plsc.VectorSubcoreMesh(core_axis_name="c", subcore_axis_name="s"),
    out_type=jax.ShapeDtypeStruct((TOTP, OUT_C), jnp.float32),
    scratch_types=(
        [pltpu.VMEM((C * N,), jnp.int32),
         pltpu.VMEM((C * N,), jnp.float32),
         pltpu.VMEM((C * N,), jnp.float32),
         pltpu.VMEM((C * N,), jnp.float32),
         pltpu.VMEM((C * N, OUT_C), jnp.float32),
         pltpu.VMEM((C, VPAD), jnp.float32),
         pltpu.VMEM((C, OUT_C), jnp.float32),
         pltpu.VMEM((C, OUT_C), jnp.float32)] * 2
        + [pltpu.VMEM((3, OUT_C), jnp.float32),
           pltpu.VMEM((1, OUT_C), jnp.float32)]
        + [pltpu.SemaphoreType.DMA] * 8
    ),
)(_sc_body)


def kernel(neighbor_index, vertices, feature_map, weights, bias, directions,
           distance_w):
    nidx = neighbor_index.astype(jnp.int32)
    offs = (jnp.arange(BS, dtype=jnp.int32) * V)[:, None, None]
    idx_flat = jnp.pad((nidx + offs).reshape(TOT * N),
                       (0, (TOTP - TOT) * N))
    vflat = vertices.reshape(TOT, 3)
    vp = jnp.pad(vflat, ((0, TOTP - TOT), (0, VPAD - 3)))
    vx = vp[:, 0]
    vy = vp[:, 1]
    vz = vp[:, 2]
    fm = jnp.pad(feature_map.reshape(TOT, IN_C), ((0, TOTP - TOT), (0, 0)))
    fc, fs = _matmul(fm, weights, bias.reshape(1, (SUP + 1) * OUT_C))
    out = _sc_kernel(idx_flat, vp, vx, vy, vz, fs, fc, directions, distance_w)
    return out[:TOT].reshape(BS, V, OUT_C)
